# baseline (speedup 1.0000x reference)
import functools

import jax
import jax.numpy as jnp
from jax import lax
from jax.experimental import pallas as pl
from jax.experimental.pallas import tpu as pltpu
from jax.experimental.pallas import tpu_sc as plsc

_NC, _NS, _L = 2, 16, 16
_NW = _NC * _NS


def _sc_gather_dmajor(w_flat, idx, d, v):
    b = idx.shape[0]
    tb = 128
    n_tok_blocks = b // tb
    n_feat_splits = _NW // n_tok_blocks
    fb = d // n_feat_splits
    mesh = plsc.VectorSubcoreMesh(core_axis_name="c", subcore_axis_name="s")

    @functools.partial(
        pl.kernel,
        mesh=mesh,
        out_type=jax.ShapeDtypeStruct((d, b), jnp.float32),
        scratch_types=[
            pltpu.VMEM((tb,), jnp.int32),
            pltpu.VMEM((fb, tb), jnp.int32),
            pltpu.VMEM((fb, tb), jnp.float32),
            pltpu.SemaphoreType.DMA,
        ],
    )
    def gather_kernel(w_hbm, idx_hbm, out_hbm, idx_v, fidx_v, rows_v, sem):
        wid = lax.axis_index("s") * _NC + lax.axis_index("c")
        k = wid % n_tok_blocks
        h = wid // n_tok_blocks
        tok0 = k * tb
        f0 = h * fb
        pltpu.sync_copy(idx_hbm.at[pl.ds(tok0, tb)], idx_v)

        def fill(f, c):
            for i in range(tb // _L):
                sl = pl.ds(i * _L, _L)
                fidx_v[f, sl] = idx_v[sl] + (f0 + f) * v
            return c

        lax.fori_loop(0, fb, fill, 0)

        def fire(f, c):
            pltpu.async_copy(w_hbm.at[fidx_v.at[f]], rows_v.at[f], sem)
            return c

        lax.fori_loop(0, fb, fire, 0)

        def drain(f, c):
            pltpu.make_async_copy(w_hbm.at[pl.ds(0, tb)], rows_v.at[f], sem).wait()
            return c

        lax.fori_loop(0, fb, drain, 0)
        pltpu.sync_copy(rows_v, out_hbm.at[pl.ds(f0, fb), pl.ds(tok0, tb)])

    return gather_kernel(w_flat, idx)


def _head_matmul_half(x_t, w_vmem, t_total, h, prev, v_blk=2048):
    d, th = x_t.shape
    _, v = w_vmem.shape
    nv = pl.cdiv(v, v_blk)

    def body(x_ref, w_ref, *rest):
        o_ref = rest[-1]
        j = pl.program_id(0)
        o_ref[...] = lax.dot_general(
            w_ref[:, pl.ds(j * v_blk, v_blk)], x_ref[...],
            dimension_numbers=(((0,), (0,)), ((), ())),
            preferred_element_type=jnp.float32,
        )

    in_specs = [
        pl.BlockSpec((d, th), lambda j: (0, 0)),
        pl.BlockSpec(memory_space=pltpu.MemorySpace.VMEM),
    ]
    operands = [x_t, w_vmem]
    aliases = {}
    if prev is not None:
        in_specs.append(pl.BlockSpec(memory_space=pl.ANY))
        operands.append(prev)
        aliases = {2: 0}

    return pl.pallas_call(
        body,
        grid=(nv,),
        in_specs=in_specs,
        out_specs=pl.BlockSpec((v_blk, th), lambda j: (j, h)),
        out_shape=jax.ShapeDtypeStruct((v, t_total), jnp.float32),
        input_output_aliases=aliases,
        compiler_params=pltpu.CompilerParams(
            dimension_semantics=("parallel",),
            vmem_limit_bytes=24 * 1024 * 1024,
        ),
    )(*operands)


def kernel(input_ids, W_emb, W_head):
    b, t = input_ids.shape
    v, d = W_emb.shape
    idx = input_ids.reshape(-1).astype(jnp.int32)
    w_flat = W_emb.T.reshape(-1)
    th = t // 2
    w_vmem = pltpu.with_memory_space_constraint(
        W_head.T, pltpu.MemorySpace.VMEM
    )
    x1 = _sc_gather_dmajor(w_flat, idx[:th], d, v)
    x2 = _sc_gather_dmajor(w_flat, idx[th:], d, v)
    logits_t = _head_matmul_half(x1, w_vmem, t, 0, None)
    logits_t = _head_matmul_half(x2, w_vmem, t, 1, logits_t)
    return logits_t.T.reshape(b, t, v)


# speedup vs baseline: 1.0471x; 1.0471x over previous
import functools

import jax
import jax.numpy as jnp
from jax import lax
from jax.experimental import pallas as pl
from jax.experimental.pallas import tpu as pltpu
from jax.experimental.pallas import tpu_sc as plsc

_NC, _NS, _L = 2, 16, 16
_NW = _NC * _NS


def _sc_gather_dmajor(w_flat, idx, d, v):
    b = idx.shape[0]
    tb = 128
    n_tok_blocks = b // tb
    n_feat_splits = _NW // n_tok_blocks
    fb = d // n_feat_splits
    mesh = plsc.VectorSubcoreMesh(core_axis_name="c", subcore_axis_name="s")

    @functools.partial(
        pl.kernel,
        mesh=mesh,
        out_type=jax.ShapeDtypeStruct((d, b), jnp.float32),
        scratch_types=[
            pltpu.VMEM((tb,), jnp.int32),
            pltpu.VMEM((fb, tb), jnp.int32),
            pltpu.VMEM((fb, tb), jnp.float32),
            pltpu.SemaphoreType.DMA,
        ],
    )
    def gather_kernel(w_hbm, idx_hbm, out_hbm, idx_v, fidx_v, rows_v, sem):
        wid = lax.axis_index("s") * _NC + lax.axis_index("c")
        k = wid % n_tok_blocks
        h = wid // n_tok_blocks
        tok0 = k * tb
        f0 = h * fb
        pltpu.sync_copy(idx_hbm.at[pl.ds(tok0, tb)], idx_v)

        def fill(f, c):
            for i in range(tb // _L):
                sl = pl.ds(i * _L, _L)
                fidx_v[f, sl] = idx_v[sl] + (f0 + f) * v
            return c

        lax.fori_loop(0, fb, fill, 0)

        def fire(f, c):
            pltpu.async_copy(w_hbm.at[fidx_v.at[f]], rows_v.at[f], sem)
            return c

        lax.fori_loop(0, fb, fire, 0)

        def drain(f, c):
            pltpu.make_async_copy(w_hbm.at[pl.ds(0, tb)], rows_v.at[f], sem).wait()
            return c

        lax.fori_loop(0, fb, drain, 0)
        pltpu.sync_copy(rows_v, out_hbm.at[pl.ds(f0, fb), pl.ds(tok0, tb)])

    return gather_kernel(w_flat, idx)


def _head_matmul_t(x_t, w_head_t, v_blk=2048):
    d, t = x_t.shape
    _, v = w_head_t.shape
    nv = pl.cdiv(v, v_blk)

    def body(x_ref, w_ref, o_ref):
        o_ref[...] = lax.dot_general(
            w_ref[...], x_ref[...],
            dimension_numbers=(((0,), (0,)), ((), ())),
            preferred_element_type=jnp.float32,
        )

    return pl.pallas_call(
        body,
        grid=(nv,),
        in_specs=[
            pl.BlockSpec((d, t), lambda j: (0, 0)),
            pl.BlockSpec((d, v_blk), lambda j: (0, j)),
        ],
        out_specs=pl.BlockSpec((v_blk, t), lambda j: (j, 0)),
        out_shape=jax.ShapeDtypeStruct((v, t), jnp.float32),
        compiler_params=pltpu.CompilerParams(
            dimension_semantics=("parallel",),
        ),
    )(x_t, w_head_t)


def kernel(input_ids, W_emb, W_head):
    b, t = input_ids.shape
    v, d = W_emb.shape
    idx = input_ids.reshape(-1).astype(jnp.int32)
    w_flat = W_emb.T.reshape(-1)
    x_t = _sc_gather_dmajor(w_flat, idx, d, v)
    logits_t = _head_matmul_t(x_t, W_head.T)
    return logits_t.T.reshape(b, t, v)
